# baseline (speedup 1.0000x reference)
import functools

import jax
import jax.numpy as jnp
from jax import lax
from jax.experimental import pallas as pl
from jax.experimental.pallas import tpu as pltpu
from jax.experimental.pallas import tpu_sc as plsc

BATCH = 16384
D = 64
NC = 2
NS = 16
NW = NC * NS
BPW = BATCH // NW
CHUNK = 128
NCHUNK = BPW // CHUNK


def _sc_gather(user, business_i, business_j, embed_user, embed_business):
    mesh = plsc.VectorSubcoreMesh(core_axis_name="c", subcore_axis_name="s")
    rows_t = jax.ShapeDtypeStruct((BATCH, D), jnp.float32)

    @functools.partial(
        pl.kernel,
        mesh=mesh,
        out_type=(rows_t, rows_t, rows_t),
        compiler_params=pltpu.CompilerParams(use_tc_tiling_on_sc=False),
        scratch_types=[
            pltpu.VMEM((BPW,), jnp.int32),
            pltpu.VMEM((BPW,), jnp.int32),
            pltpu.VMEM((BPW,), jnp.int32),
            pltpu.VMEM((BPW, D), jnp.float32),
            pltpu.VMEM((BPW, D), jnp.float32),
            pltpu.VMEM((BPW, D), jnp.float32),
            pltpu.SemaphoreType.DMA,
        ],
    )
    def k(u_hbm, bi_hbm, bj_hbm, eu_hbm, eb_hbm,
          ou_hbm, obi_hbm, obj_hbm,
          uidx, biidx, bjidx, urows, birows, bjrows, sem):
        wid = lax.axis_index("s") * NC + lax.axis_index("c")
        base = wid * BPW
        pltpu.sync_copy(u_hbm.at[pl.ds(base, BPW)], uidx)
        pltpu.sync_copy(bi_hbm.at[pl.ds(base, BPW)], biidx)
        pltpu.sync_copy(bj_hbm.at[pl.ds(base, BPW)], bjidx)
        copies = []
        for table, idx, rows in ((eu_hbm, uidx, urows),
                                 (eb_hbm, biidx, birows),
                                 (eb_hbm, bjidx, bjrows)):
            for c in range(NCHUNK):
                sl = pl.ds(c * CHUNK, CHUNK)
                copies.append(
                    pltpu.async_copy(table.at[idx.at[sl]], rows.at[sl], sem))
        for cp in copies:
            cp.wait()
        pltpu.sync_copy(urows, ou_hbm.at[pl.ds(base, BPW)])
        pltpu.sync_copy(birows, obi_hbm.at[pl.ds(base, BPW)])
        pltpu.sync_copy(bjrows, obj_hbm.at[pl.ds(base, BPW)])

    return k(user, business_i, business_j, embed_user, embed_business)


TC_ROWS = 2048


def _tc_body(u_ref, bi_ref, bj_ref, oi_ref, oj_ref):
    u = u_ref[...]
    oi_ref[...] = jnp.sum(u * bi_ref[...], axis=1, keepdims=True)
    oj_ref[...] = jnp.sum(u * bj_ref[...], axis=1, keepdims=True)


def _tc_reduce(urows, birows, bjrows):
    out_t = jax.ShapeDtypeStruct((BATCH, 1), jnp.float32)
    grid = (BATCH // TC_ROWS,)
    in_spec = pl.BlockSpec((TC_ROWS, D), lambda i: (i, 0))
    out_spec = pl.BlockSpec((TC_ROWS, 1), lambda i: (i, 0))
    return pl.pallas_call(
        _tc_body,
        grid=grid,
        in_specs=[in_spec, in_spec, in_spec],
        out_specs=(out_spec, out_spec),
        out_shape=(out_t, out_t),
    )(urows, birows, bjrows)


@jax.jit
def kernel(user, business_i, business_j, embed_user, embed_business):
    urows, birows, bjrows = _sc_gather(
        user, business_i, business_j, embed_user, embed_business)
    pi, pj = _tc_reduce(urows, birows, bjrows)
    return pi.reshape(BATCH), pj.reshape(BATCH)
